# Initial kernel scaffold; baseline (speedup 1.0000x reference)
import jax
import jax.numpy as jnp
from jax.experimental import pallas as pl


def kernel(action, prefix_embs, prefix_pad_masks, prefix_att_masks, states, task_ids, Wq, Wk, Wv, Wo, W1, b1, W2, b2):
    raise NotImplementedError("write your pallas kernel here")


# speedup vs baseline: 1.2567x; 1.2567x over previous
import functools
import numpy as np
import jax
import jax.numpy as jnp
from jax import lax
from jax.experimental import pallas as pl
from jax.experimental.pallas import tpu as pltpu

CN, HN, AH = 4, 2, 8
B, L, D, S, A, HID = 1024, 20, 256, 256, 64, 256
SA = S + A
DH = D // AH
R = 64
NBLK = B // R + CN
G = NBLK * R


def _compute_body(cid_ref, qin_ref, pre_ref, wq_ref, wk_ref, wv_ref,
                  wo_ref, w1_ref, b1_ref, w2_ref, b2_ref, out_ref):
    pre = pre_ref[...]
    qin = qin_ref[...]
    d_i = lax.broadcasted_iota(jnp.int32, (D, AH), 0)
    a_i = lax.broadcasted_iota(jnp.int32, (D, AH), 1)
    E = (d_i // DH == a_i).astype(jnp.float32)
    scale = 1.0 / np.sqrt(DH)
    qvs = []
    for h in range(HN):
        q = jnp.dot(qin, wq_ref[0, h], preferred_element_type=jnp.float32)
        k = jnp.dot(pre, wk_ref[0, h], preferred_element_type=jnp.float32)
        v = jnp.dot(pre, wv_ref[0, h], preferred_element_type=jnp.float32)
        prod = k.reshape(R, L, D) * q.reshape(R, 1, D)
        scores = jnp.dot(prod.reshape(R * L, D), E,
                         preferred_element_type=jnp.float32) * scale
        s3 = scores.reshape(R, L, AH)
        m = jnp.max(s3, axis=1, keepdims=True)
        e = jnp.exp(s3 - m)
        attn = e / jnp.sum(e, axis=1, keepdims=True)
        attn_exp = jnp.dot(attn.reshape(R * L, AH), E.T,
                           preferred_element_type=jnp.float32)
        ctx = jnp.sum((attn_exp * v).reshape(R, L, D), axis=1)
        out = jnp.dot(ctx, wo_ref[0, h], preferred_element_type=jnp.float32)
        hid = jnp.maximum(jnp.dot(out, w1_ref[0, h],
                                  preferred_element_type=jnp.float32)
                          + b1_ref[0, h][None, :], 0.0)
        qv = jnp.sum(hid * w2_ref[0, h][None, :], axis=-1) + b2_ref[0, h]
        qvs.append(qv)
    lane = lax.broadcasted_iota(jnp.int32, (R, 16), 1)
    out_ref[...] = (jnp.where(lane == 0, qvs[0][:, None], 0.0)
                    + jnp.where(lane == 1, qvs[1][:, None], 0.0))


def _moe_compute(block_cid, qin_g, pre_g, Wq, Wk, Wv, Wo, W1, b1, W2s, b2s):
    grid_spec = pltpu.PrefetchScalarGridSpec(
        num_scalar_prefetch=1,
        grid=(NBLK,),
        in_specs=[
            pl.BlockSpec((R, SA), lambda i, cid: (i, 0)),
            pl.BlockSpec((R * L, D), lambda i, cid: (i, 0)),
            pl.BlockSpec((1, HN, SA, D), lambda i, cid: (cid[i], 0, 0, 0)),
            pl.BlockSpec((1, HN, D, D), lambda i, cid: (cid[i], 0, 0, 0)),
            pl.BlockSpec((1, HN, D, D), lambda i, cid: (cid[i], 0, 0, 0)),
            pl.BlockSpec((1, HN, D, D), lambda i, cid: (cid[i], 0, 0, 0)),
            pl.BlockSpec((1, HN, D, HID), lambda i, cid: (cid[i], 0, 0, 0)),
            pl.BlockSpec((1, HN, HID), lambda i, cid: (cid[i], 0, 0)),
            pl.BlockSpec((1, HN, HID), lambda i, cid: (cid[i], 0, 0)),
            pl.BlockSpec((1, HN, 1), lambda i, cid: (cid[i], 0, 0)),
        ],
        out_specs=pl.BlockSpec((R, 16), lambda i, cid: (i, 0)),
    )
    return pl.pallas_call(
        _compute_body,
        grid_spec=grid_spec,
        out_shape=jax.ShapeDtypeStruct((G, 16), jnp.float32),
    )(block_cid, qin_g, pre_g, Wq, Wk, Wv, Wo, W1, b1, W2s, b2s)


def _route_jnp(task_ids):
    cids = jnp.remainder(task_ids.astype(jnp.int32), CN)
    counts = jnp.sum(cids[None, :] == jnp.arange(CN, dtype=jnp.int32)[:, None],
                     axis=1)
    aligned = ((counts + R - 1) // R) * R
    starts = jnp.concatenate([jnp.zeros((1,), jnp.int32),
                              jnp.cumsum(aligned)[:-1].astype(jnp.int32)])
    gstarts = jnp.concatenate([jnp.zeros((1,), jnp.int32),
                               jnp.cumsum(counts)[:-1].astype(jnp.int32)])
    order = jnp.argsort(cids, stable=True).astype(jnp.int32)
    cs = cids[order]
    rank = jnp.arange(B, dtype=jnp.int32) - gstarts[cs]
    pos_sorted = starts[cs] + rank
    pos = jnp.zeros((B,), jnp.int32).at[order].set(pos_sorted)
    perm = jnp.zeros((G,), jnp.int32).at[pos_sorted].set(order)
    ends = jnp.cumsum(aligned).astype(jnp.int32)
    bs = jnp.arange(NBLK, dtype=jnp.int32) * R
    bcid = jnp.sum(bs[None, :] >= ends[:CN - 1, None], axis=0).astype(jnp.int32)
    return perm, pos, bcid


def kernel(action, prefix_embs, prefix_pad_masks, prefix_att_masks, states,
           task_ids, Wq, Wk, Wv, Wo, W1, b1, W2, b2):
    qin = jnp.concatenate([states, action], axis=-1)
    perm, pos, bcid = _route_jnp(task_ids)
    qin_g = qin[perm]
    pre_g = prefix_embs.reshape(B, L * D)[perm]
    outs = _moe_compute(bcid, qin_g, pre_g.reshape(G * L, D),
                        Wq, Wk, Wv, Wo, W1, b1,
                        W2.reshape(CN, HN, HID), b2)
    return outs[pos][:, :2]
